# Initial kernel scaffold; baseline (speedup 1.0000x reference)
import jax
import jax.numpy as jnp
from jax.experimental import pallas as pl


def kernel(inputs, domain_idx):
    raise NotImplementedError("write your pallas kernel here")


# speedup vs baseline: 11.7888x; 11.7888x over previous
import jax
import jax.numpy as jnp
from jax.experimental import pallas as pl
from jax.experimental.pallas import tpu as pltpu

_DOMAIN_TO_CHOSEN = ((0, 2, 5), (1, 3, 6), (2, 4, 7), (0, 1, 2))

_ROWS_PER_BLOCK = 512


def _mean3_body(chosen_ref, x0_ref, x1_ref, x2_ref, o_ref):
    del chosen_ref
    o_ref[...] = (x0_ref[0] + x1_ref[0] + x2_ref[0]) * jnp.float32(1.0 / 3.0)


def kernel(inputs, domain_idx):
    n_cand, n_rows, n_cols = inputs.shape
    table = jnp.asarray(_DOMAIN_TO_CHOSEN, dtype=jnp.int32)
    chosen = table[domain_idx]

    nb = n_rows // _ROWS_PER_BLOCK
    blk = (1, _ROWS_PER_BLOCK, n_cols)

    def in_spec(k):
        return pl.BlockSpec(blk, lambda i, cref, _k=k: (cref[_k], i, 0))

    grid_spec = pltpu.PrefetchScalarGridSpec(
        num_scalar_prefetch=1,
        grid=(nb,),
        in_specs=[in_spec(0), in_spec(1), in_spec(2)],
        out_specs=pl.BlockSpec((_ROWS_PER_BLOCK, n_cols), lambda i, cref: (i, 0)),
    )

    return pl.pallas_call(
        _mean3_body,
        grid_spec=grid_spec,
        out_shape=jax.ShapeDtypeStruct((n_rows, n_cols), inputs.dtype),
    )(chosen, inputs, inputs, inputs)


# speedup vs baseline: 11.8652x; 1.0065x over previous
import jax
import jax.numpy as jnp
from jax.experimental import pallas as pl
from jax.experimental.pallas import tpu as pltpu

_DOMAIN_TO_CHOSEN = ((0, 2, 5), (1, 3, 6), (2, 4, 7), (0, 1, 2))

_ROWS_PER_BLOCK = 256


def _mean3_body(chosen_ref, x0_ref, x1_ref, x2_ref, o_ref):
    del chosen_ref
    o_ref[...] = (x0_ref[0] + x1_ref[0] + x2_ref[0]) * jnp.float32(1.0 / 3.0)


def kernel(inputs, domain_idx):
    n_cand, n_rows, n_cols = inputs.shape
    table = jnp.asarray(_DOMAIN_TO_CHOSEN, dtype=jnp.int32)
    chosen = table[domain_idx]

    nb = n_rows // _ROWS_PER_BLOCK
    blk = (1, _ROWS_PER_BLOCK, n_cols)

    def in_spec(k):
        return pl.BlockSpec(blk, lambda i, cref, _k=k: (cref[_k], i, 0))

    grid_spec = pltpu.PrefetchScalarGridSpec(
        num_scalar_prefetch=1,
        grid=(nb,),
        in_specs=[in_spec(0), in_spec(1), in_spec(2)],
        out_specs=pl.BlockSpec((_ROWS_PER_BLOCK, n_cols), lambda i, cref: (i, 0)),
    )

    return pl.pallas_call(
        _mean3_body,
        grid_spec=grid_spec,
        out_shape=jax.ShapeDtypeStruct((n_rows, n_cols), inputs.dtype),
    )(chosen, inputs, inputs, inputs)
